# Initial kernel scaffold; baseline (speedup 1.0000x reference)
import jax
import jax.numpy as jnp
from jax.experimental import pallas as pl


def kernel(node_feats, edge_feats, edge_index, W_node, gamma_node, beta_node, W_edge, gamma_edge, beta_edge, W_out, gamma_out, beta_out):
    raise NotImplementedError("write your pallas kernel here")


# speedup vs baseline: 43.7863x; 43.7863x over previous
import functools

import jax
import jax.numpy as jnp
from jax import lax
from jax.experimental import pallas as pl
from jax.experimental.pallas import tpu as pltpu
from jax.experimental.pallas import tpu_sc as plsc

_N = 10000
_E = 320000
_D = 128

_NC = 2
_NS = 16
_NW = _NC * _NS
_EPW = _E // _NW
_C = 80
_NCHUNK = _EPW // _C
_RPS = 624
_TAIL = _N - _NS * _RPS


def _gelu(x):
    return 0.5 * x * (1.0 + lax.erf(x * (2.0 ** -0.5)))


def _ln(y, gamma, beta):
    mu = jnp.mean(y, axis=-1, keepdims=True)
    var = jnp.mean((y - mu) ** 2, axis=-1, keepdims=True)
    return (y - mu) / jnp.sqrt(var + 1e-5) * gamma + beta


def _node_proj_body(x_ref, w_ref, g_ref, b_ref, o_ref):
    y = jnp.dot(x_ref[...], w_ref[...], preferred_element_type=jnp.float32)
    y = _gelu(y)
    o_ref[...] = _ln(y, g_ref[...], b_ref[...])


def _edge_proj_body(x_ref, w_ref, g_ref, b_ref, o_ref):
    y = jnp.dot(x_ref[...], w_ref[...], preferred_element_type=jnp.float32)
    o_ref[...] = jnp.exp(_ln(y, g_ref[...], b_ref[...]))


def _out_proj_body(h0_ref, h1_ref, w_ref, g_ref, b_ref, o_ref):
    h = h0_ref[...] + h1_ref[...]
    y = jnp.dot(h, w_ref[...], preferred_element_type=jnp.float32)
    y = _gelu(y)
    o_ref[...] = _ln(y, g_ref[...], b_ref[...])


def _node_proj(x, w, g, b, rows_blk):
    n = x.shape[0]
    k = x.shape[1]
    d = w.shape[1]
    grid = n // rows_blk
    return pl.pallas_call(
        _node_proj_body,
        grid=(grid,),
        in_specs=[
            pl.BlockSpec((rows_blk, k), lambda i: (i, 0)),
            pl.BlockSpec((k, d), lambda i: (0, 0)),
            pl.BlockSpec((1, d), lambda i: (0, 0)),
            pl.BlockSpec((1, d), lambda i: (0, 0)),
        ],
        out_specs=pl.BlockSpec((rows_blk, d), lambda i: (i, 0)),
        out_shape=jax.ShapeDtypeStruct((n, d), jnp.float32),
    )(x, w, g.reshape(1, d), b.reshape(1, d))


def _edge_proj(x, w, g, b, rows_blk):
    n = x.shape[0]
    k = x.shape[1]
    d = w.shape[1]
    grid = n // rows_blk
    return pl.pallas_call(
        _edge_proj_body,
        grid=(grid,),
        in_specs=[
            pl.BlockSpec((rows_blk, k), lambda i: (i, 0)),
            pl.BlockSpec((k, d), lambda i: (0, 0)),
            pl.BlockSpec((1, d), lambda i: (0, 0)),
            pl.BlockSpec((1, d), lambda i: (0, 0)),
        ],
        out_specs=pl.BlockSpec((rows_blk, d), lambda i: (i, 0)),
        out_shape=jax.ShapeDtypeStruct((n, d), jnp.float32),
    )(x, w, g.reshape(1, d), b.reshape(1, d))


def _out_proj(h2, w, g, b, rows_blk):
    n = h2.shape[0] // 2
    d = h2.shape[1]
    dout = w.shape[1]
    grid = n // rows_blk
    nblk = grid
    return pl.pallas_call(
        _out_proj_body,
        grid=(grid,),
        in_specs=[
            pl.BlockSpec((rows_blk, d), lambda i: (i, 0)),
            pl.BlockSpec((rows_blk, d), lambda i, _n=nblk: (i + _n, 0)),
            pl.BlockSpec((d, dout), lambda i: (0, 0)),
            pl.BlockSpec((1, dout), lambda i: (0, 0)),
            pl.BlockSpec((1, dout), lambda i: (0, 0)),
        ],
        out_specs=pl.BlockSpec((rows_blk, dout), lambda i: (i, 0)),
        out_shape=jax.ShapeDtypeStruct((n, dout), jnp.float32),
    )(h2, h2, w, g.reshape(1, dout), b.reshape(1, dout))


def _sc_body(hv_hbm, he_hbm, src_hbm, dst_hbm, zero_hbm, out_hbm,
             srcv, dstv, hev, hvv, hsh, sem):
    cid = lax.axis_index("c")
    sid = lax.axis_index("s")
    wid = sid * _NC + cid

    pltpu.sync_copy(zero_hbm.at[pl.ds(sid * _RPS, _RPS)],
                    hsh.at[pl.ds(sid * _RPS, _RPS)])

    @pl.when(sid == 0)
    def _zero_tail():
        pltpu.sync_copy(zero_hbm.at[pl.ds(_NS * _RPS, _TAIL)],
                        hsh.at[pl.ds(_NS * _RPS, _TAIL)])

    plsc.subcore_barrier()

    base = wid * _EPW

    def chunk(j, carry):
        off = base + j * _C
        pltpu.sync_copy(src_hbm.at[pl.ds(off, _C)], srcv)
        pltpu.sync_copy(dst_hbm.at[pl.ds(off, _C)], dstv)
        cp_he = pltpu.async_copy(he_hbm.at[pl.ds(off, _C)], hev, sem)
        cp_hv = pltpu.async_copy(hv_hbm.at[srcv], hvv, sem)
        cp_he.wait()
        cp_hv.wait()

        def mul_row(r, c2):
            for k in range(_D // 16):
                sl = pl.ds(k * 16, 16)
                hvv[r, sl] = hvv[r, sl] * hev[r, sl]
            return c2

        lax.fori_loop(0, _C, mul_row, 0)
        pltpu.sync_copy(hvv, hsh.at[dstv], add=True)
        return carry

    lax.fori_loop(0, _NCHUNK, chunk, 0)

    plsc.subcore_barrier()
    pltpu.sync_copy(hsh.at[pl.ds(sid * _RPS, _RPS)],
                    out_hbm.at[pl.ds(cid * _N + sid * _RPS, _RPS)])

    @pl.when(sid == 0)
    def _out_tail():
        pltpu.sync_copy(hsh.at[pl.ds(_NS * _RPS, _TAIL)],
                        out_hbm.at[pl.ds(cid * _N + _NS * _RPS, _TAIL)])


_sc_aggregate = functools.partial(
    pl.kernel,
    out_type=jax.ShapeDtypeStruct((2 * _N, _D), jnp.float32),
    mesh=plsc.VectorSubcoreMesh(core_axis_name="c", subcore_axis_name="s"),
    scratch_types=[
        pltpu.VMEM((_C,), jnp.int32),
        pltpu.VMEM((_C,), jnp.int32),
        pltpu.VMEM((_C, _D), jnp.float32),
        pltpu.VMEM((_C, _D), jnp.float32),
        pltpu.VMEM_SHARED((_N, _D), jnp.float32),
        pltpu.SemaphoreType.DMA,
    ],
)(_sc_body)


def kernel(node_feats, edge_feats, edge_index, W_node, gamma_node, beta_node,
           W_edge, gamma_edge, beta_edge, W_out, gamma_out, beta_out):
    hv = _node_proj(node_feats, W_node, gamma_node, beta_node, rows_blk=1000)
    he = _edge_proj(edge_feats, W_edge, gamma_edge, beta_edge, rows_blk=2000)
    src = edge_index[0].astype(jnp.int32)
    dst = edge_index[1].astype(jnp.int32)
    zero = jnp.zeros((_N, _D), jnp.float32)
    h2 = _sc_aggregate(hv, he, src, dst, zero)
    return _out_proj(h2, W_out, gamma_out, beta_out, rows_blk=1000)


# speedup vs baseline: 58.4873x; 1.3357x over previous
import functools

import jax
import jax.numpy as jnp
from jax import lax
from jax.experimental import pallas as pl
from jax.experimental.pallas import tpu as pltpu
from jax.experimental.pallas import tpu_sc as plsc

_N = 10000
_E = 320000
_D = 128

_NC = 2
_NS = 16
_NW = _NC * _NS
_EPW = _E // _NW
_C = 40
_NCHUNK = _EPW // _C
_STEP = 12
_MAIN = (_NCHUNK - 10) // _STEP * _STEP
_RPS = 624
_TAIL = _N - _NS * _RPS


def _gelu(x):
    return 0.5 * x * (1.0 + lax.erf(x * (2.0 ** -0.5)))


def _ln(y, gamma, beta):
    mu = jnp.mean(y, axis=-1, keepdims=True)
    var = jnp.mean((y - mu) ** 2, axis=-1, keepdims=True)
    return (y - mu) / jnp.sqrt(var + 1e-5) * gamma + beta


def _node_proj_body(x_ref, w_ref, g_ref, b_ref, o_ref):
    y = jnp.dot(x_ref[...], w_ref[...], preferred_element_type=jnp.float32)
    y = _gelu(y)
    o_ref[...] = _ln(y, g_ref[...], b_ref[...])


def _edge_proj_body(x_ref, w_ref, g_ref, b_ref, o_ref):
    y = jnp.dot(x_ref[...], w_ref[...], preferred_element_type=jnp.float32)
    o_ref[...] = jnp.exp(_ln(y, g_ref[...], b_ref[...]))


def _out_proj_body(h0_ref, h1_ref, w_ref, g_ref, b_ref, o_ref):
    h = h0_ref[...] + h1_ref[...]
    y = jnp.dot(h, w_ref[...], preferred_element_type=jnp.float32)
    y = _gelu(y)
    o_ref[...] = _ln(y, g_ref[...], b_ref[...])


def _dense_proj(body, x_parts, w, g, b, rows_blk):
    n = x_parts[0].shape[0] if len(x_parts) == 1 else x_parts[0].shape[0] // 2
    k = x_parts[0].shape[1]
    d = w.shape[1]
    nblk = n // rows_blk
    if len(x_parts) == 1:
        in_specs = [pl.BlockSpec((rows_blk, k), lambda i: (i, 0))]
        args = list(x_parts)
    else:
        in_specs = [
            pl.BlockSpec((rows_blk, k), lambda i: (i, 0)),
            pl.BlockSpec((rows_blk, k), lambda i, _n=nblk: (i + _n, 0)),
        ]
        args = list(x_parts)
    return pl.pallas_call(
        body,
        grid=(nblk,),
        in_specs=in_specs + [
            pl.BlockSpec((k, d), lambda i: (0, 0)),
            pl.BlockSpec((1, d), lambda i: (0, 0)),
            pl.BlockSpec((1, d), lambda i: (0, 0)),
        ],
        out_specs=pl.BlockSpec((rows_blk, d), lambda i: (i, 0)),
        out_shape=jax.ShapeDtypeStruct((n, d), jnp.float32),
    )(*args, w, g.reshape(1, d), b.reshape(1, d))


def _sc_body(hv_hbm, he_hbm, src_hbm, dst_hbm, zero_hbm, out_hbm,
             srcv0, srcv1, dstv0, dstv1, dstv2, dstv3,
             hev0, hev1, hvv0, hvv1, hvv2, hsh,
             semidx0, semidx1, semhe0, semhe1,
             semgv0, semgv1, semgv2, semsc0, semsc1, semsc2):
    srcv = (srcv0, srcv1)
    dstv = (dstv0, dstv1, dstv2, dstv3)
    hev = (hev0, hev1)
    hvv = (hvv0, hvv1, hvv2)
    semidx = (semidx0, semidx1)
    semhe = (semhe0, semhe1)
    semgv = (semgv0, semgv1, semgv2)
    semsc = (semsc0, semsc1, semsc2)

    cid = lax.axis_index("c")
    sid = lax.axis_index("s")
    wid = sid * _NC + cid
    base = wid * _EPW

    pltpu.sync_copy(zero_hbm.at[pl.ds(sid * _RPS, _RPS)],
                    hsh.at[pl.ds(sid * _RPS, _RPS)])

    @pl.when(sid == 0)
    def _zero_tail():
        pltpu.sync_copy(zero_hbm.at[pl.ds(_NS * _RPS, _TAIL)],
                        hsh.at[pl.ds(_NS * _RPS, _TAIL)])

    plsc.subcore_barrier()

    def issue_idx(g, s, d4):
        off = pl.ds(base + g * _C, _C)
        pltpu.async_copy(src_hbm.at[off], srcv[s], semidx[s])
        pltpu.async_copy(dst_hbm.at[off], dstv[d4], semidx[s])

    def wait_idx(s, d4):
        pltpu.make_async_copy(src_hbm.at[pl.ds(0, _C)], srcv[s],
                              semidx[s]).wait()
        pltpu.make_async_copy(src_hbm.at[pl.ds(0, _C)], dstv[d4],
                              semidx[s]).wait()

    def issue_he(g, e):
        pltpu.async_copy(he_hbm.at[pl.ds(base + g * _C, _C)], hev[e], semhe[e])

    def issue_gather(s, v):
        pltpu.async_copy(hv_hbm.at[srcv[s]], hvv[v], semgv[v])

    def drain_scatter(v):
        pltpu.make_async_copy(he_hbm.at[pl.ds(0, _C)], hvv[v], semsc[v]).wait()

    def chunk_body(g, q, static):
        v = q % 3
        e = q % 2
        s = q % 2
        d4 = q % 4
        vn = (q + 1) % 3
        sn = (q + 1) % 2

        if static:
            if g >= 2:
                drain_scatter(vn)
        else:
            @pl.when(g >= 2)
            def _drain():
                drain_scatter(vn)

        if (not static) or g + 1 < _NCHUNK:
            wait_idx(sn, (q + 1) % 4)
            issue_gather(sn, vn)

        pltpu.make_async_copy(he_hbm.at[pl.ds(0, _C)], hev[e],
                              semhe[e]).wait()
        pltpu.make_async_copy(he_hbm.at[pl.ds(0, _C)], hvv[v],
                              semgv[v]).wait()

        @plsc.parallel_loop(0, _C, 1, unroll=2)
        def _mul(r):
            for k in range(_D // 16):
                sl = pl.ds(k * 16, 16)
                hvv[v][r, sl] = hvv[v][r, sl] * hev[e][r, sl]

        if static and g == _NCHUNK - 1:
            pltpu.sync_copy(hvv[v], hsh.at[dstv[d4]], add=True)
        else:
            pltpu.async_copy(hvv[v], hsh.at[dstv[d4]], semsc[v], add=True)

        if (not static) or g + 2 < _NCHUNK:
            issue_idx(g + 2, s, (q + 2) % 4)
            issue_he(g + 2, e)

    issue_idx(0, 0, 0)
    issue_idx(1, 1, 1)
    issue_he(0, 0)
    issue_he(1, 1)
    wait_idx(0, 0)
    issue_gather(0, 0)

    @pl.loop(0, _MAIN, step=_STEP)
    def _main(j):
        for q in range(_STEP):
            chunk_body(j + q, q, False)

    for g in range(_MAIN, _NCHUNK):
        chunk_body(g, g - _MAIN, True)

    drain_scatter((_NCHUNK - 2) % 3)

    plsc.subcore_barrier()
    pltpu.sync_copy(hsh.at[pl.ds(sid * _RPS, _RPS)],
                    out_hbm.at[pl.ds(cid * _N + sid * _RPS, _RPS)])

    @pl.when(sid == 0)
    def _out_tail():
        pltpu.sync_copy(hsh.at[pl.ds(_NS * _RPS, _TAIL)],
                        out_hbm.at[pl.ds(cid * _N + _NS * _RPS, _TAIL)])


_sc_aggregate = functools.partial(
    pl.kernel,
    out_type=jax.ShapeDtypeStruct((2 * _N, _D), jnp.float32),
    mesh=plsc.VectorSubcoreMesh(core_axis_name="c", subcore_axis_name="s"),
    scratch_types=[
        pltpu.VMEM((_C,), jnp.int32),
        pltpu.VMEM((_C,), jnp.int32),
        pltpu.VMEM((_C,), jnp.int32),
        pltpu.VMEM((_C,), jnp.int32),
        pltpu.VMEM((_C,), jnp.int32),
        pltpu.VMEM((_C,), jnp.int32),
        pltpu.VMEM((_C, _D), jnp.float32),
        pltpu.VMEM((_C, _D), jnp.float32),
        pltpu.VMEM((_C, _D), jnp.float32),
        pltpu.VMEM((_C, _D), jnp.float32),
        pltpu.VMEM((_C, _D), jnp.float32),
        pltpu.VMEM_SHARED((_N, _D), jnp.float32),
        pltpu.SemaphoreType.DMA,
        pltpu.SemaphoreType.DMA,
        pltpu.SemaphoreType.DMA,
        pltpu.SemaphoreType.DMA,
        pltpu.SemaphoreType.DMA,
        pltpu.SemaphoreType.DMA,
        pltpu.SemaphoreType.DMA,
        pltpu.SemaphoreType.DMA,
        pltpu.SemaphoreType.DMA,
        pltpu.SemaphoreType.DMA,
    ],
)(_sc_body)


def kernel(node_feats, edge_feats, edge_index, W_node, gamma_node, beta_node,
           W_edge, gamma_edge, beta_edge, W_out, gamma_out, beta_out):
    hv = _dense_proj(_node_proj_body, [node_feats], W_node, gamma_node,
                     beta_node, rows_blk=1000)
    he = _dense_proj(_edge_proj_body, [edge_feats], W_edge, gamma_edge,
                     beta_edge, rows_blk=2000)
    src = edge_index[0].astype(jnp.int32)
    dst = edge_index[1].astype(jnp.int32)
    zero = jnp.zeros((_N, _D), jnp.float32)
    h2 = _sc_aggregate(hv, he, src, dst, zero)
    return _dense_proj(_out_proj_body, [h2, h2], W_out, gamma_out, beta_out,
                       rows_blk=1000)


# speedup vs baseline: 82.8860x; 1.4172x over previous
import functools

import jax
import jax.numpy as jnp
from jax import lax
from jax.experimental import pallas as pl
from jax.experimental.pallas import tpu as pltpu
from jax.experimental.pallas import tpu_sc as plsc

_N = 10000
_E = 320000
_D = 128

_NC = 2
_NS = 16
_NW = _NC * _NS
_EPW = _E // _NW
_C = 40
_NCHUNK = _EPW // _C
_STEP = 12
_MAIN = (_NCHUNK - 10) // _STEP * _STEP
_RPS = 624
_TAIL = _N - _NS * _RPS


def _gelu(x):
    return 0.5 * x * (1.0 + lax.erf(x * (2.0 ** -0.5)))


def _ln(y, gamma, beta):
    mu = jnp.mean(y, axis=-1, keepdims=True)
    var = jnp.mean((y - mu) ** 2, axis=-1, keepdims=True)
    return (y - mu) / jnp.sqrt(var + 1e-5) * gamma + beta


def _node_proj_body(x_ref, w_ref, g_ref, b_ref, ei_ref, o_ref, src_ref,
                    dst_ref):
    y = jnp.dot(x_ref[...], w_ref[...], preferred_element_type=jnp.float32)
    y = _gelu(y)
    o_ref[...] = _ln(y, g_ref[...], b_ref[...])
    @pl.when(pl.program_id(0) == 0)
    def _emit_idx():
        ei = ei_ref[...]
        src_ref[...] = ei[0, :]
        dst_ref[...] = ei[1, :]


def _edge_proj_body(xt_ref, w_ref, g_ref, b_ref, o_ref):
    y = lax.dot_general(xt_ref[...], w_ref[...], (((0,), (0,)), ((), ())),
                        preferred_element_type=jnp.float32)
    d = y.shape[1]
    c = jnp.full((d, 1), 1.0 / d, jnp.float32)
    mu = jnp.dot(y, c, preferred_element_type=jnp.float32)
    m2 = jnp.dot(y * y, c, preferred_element_type=jnp.float32)
    var = m2 - mu * mu
    rstd = lax.rsqrt(var + 1e-5)
    o_ref[...] = jnp.exp((y - mu) * rstd * g_ref[...] + b_ref[...])


def _out_proj_body(h0_ref, h1_ref, w_ref, g_ref, b_ref, o_ref):
    h = h0_ref[...] + h1_ref[...]
    y = jnp.dot(h, w_ref[...], preferred_element_type=jnp.float32)
    y = _gelu(y)
    o_ref[...] = _ln(y, g_ref[...], b_ref[...])


def _edge_proj(x, w, g, b, cols_blk):
    xt = x.T
    k, n = xt.shape
    d = w.shape[1]
    nblk = n // cols_blk
    return pl.pallas_call(
        _edge_proj_body,
        grid=(nblk,),
        in_specs=[
            pl.BlockSpec((k, cols_blk), lambda i: (0, i)),
            pl.BlockSpec((k, d), lambda i: (0, 0)),
            pl.BlockSpec((1, d), lambda i: (0, 0)),
            pl.BlockSpec((1, d), lambda i: (0, 0)),
        ],
        out_specs=pl.BlockSpec((cols_blk, d), lambda i: (i, 0)),
        out_shape=jax.ShapeDtypeStruct((n, d), jnp.float32),
    )(xt, w, g.reshape(1, d), b.reshape(1, d))


def _node_proj(x, w, g, b, edge_index, rows_blk):
    n, k = x.shape
    d = w.shape[1]
    nblk = n // rows_blk
    return pl.pallas_call(
        _node_proj_body,
        grid=(nblk,),
        in_specs=[
            pl.BlockSpec((rows_blk, k), lambda i: (i, 0)),
            pl.BlockSpec((k, d), lambda i: (0, 0)),
            pl.BlockSpec((1, d), lambda i: (0, 0)),
            pl.BlockSpec((1, d), lambda i: (0, 0)),
            pl.BlockSpec((2, _E), lambda i: (0, 0)),
        ],
        out_specs=[
            pl.BlockSpec((rows_blk, d), lambda i: (i, 0)),
            pl.BlockSpec((_E,), lambda i: (0,)),
            pl.BlockSpec((_E,), lambda i: (0,)),
        ],
        out_shape=[
            jax.ShapeDtypeStruct((n, d), jnp.float32),
            jax.ShapeDtypeStruct((_E,), jnp.int32),
            jax.ShapeDtypeStruct((_E,), jnp.int32),
        ],
    )(x, w, g.reshape(1, d), b.reshape(1, d), edge_index)


def _dense_proj(body, x_parts, w, g, b, rows_blk):
    n = x_parts[0].shape[0] // 2
    k = x_parts[0].shape[1]
    d = w.shape[1]
    nblk = n // rows_blk
    in_specs = [
        pl.BlockSpec((rows_blk, k), lambda i: (i, 0)),
        pl.BlockSpec((rows_blk, k), lambda i, _n=nblk: (i + _n, 0)),
    ]
    args = list(x_parts)
    return pl.pallas_call(
        body,
        grid=(nblk,),
        in_specs=in_specs + [
            pl.BlockSpec((k, d), lambda i: (0, 0)),
            pl.BlockSpec((1, d), lambda i: (0, 0)),
            pl.BlockSpec((1, d), lambda i: (0, 0)),
        ],
        out_specs=pl.BlockSpec((rows_blk, d), lambda i: (i, 0)),
        out_shape=jax.ShapeDtypeStruct((n, d), jnp.float32),
    )(*args, w, g.reshape(1, d), b.reshape(1, d))


def _make_sc_aggregate(epw):
    nchunk = epw // _C
    main = (nchunk - 2) // _STEP * _STEP

    def _sc_body(ebase0, hv_hbm, he_hbm, src_hbm, dst_hbm, init_hbm, out_hbm,
                 srcv0, srcv1, dstv0, dstv1, dstv2, dstv3,
                 hev0, hev1, hvv0, hvv1, hvv2, hsh,
                 semidx0, semidx1, semhe0, semhe1,
                 semgv0, semgv1, semgv2, semsc0, semsc1, semsc2):
        srcv = (srcv0, srcv1)
        dstv = (dstv0, dstv1, dstv2, dstv3)
        hev = (hev0, hev1)
        hvv = (hvv0, hvv1, hvv2)
        semidx = (semidx0, semidx1)
        semhe = (semhe0, semhe1)
        semgv = (semgv0, semgv1, semgv2)
        semsc = (semsc0, semsc1, semsc2)

        cid = lax.axis_index("c")
        sid = lax.axis_index("s")
        wid = sid * _NC + cid
        base = wid * epw
        ibase = ebase0 + base

        def issue_idx(g, s, d4):
            off = pl.ds(ibase + g * _C, _C)
            pltpu.async_copy(src_hbm.at[off], srcv[s], semidx[s])
            pltpu.async_copy(dst_hbm.at[off], dstv[d4], semidx[s])

        def wait_idx(s, d4):
            pltpu.make_async_copy(src_hbm.at[pl.ds(0, _C)], srcv[s],
                                  semidx[s]).wait()
            pltpu.make_async_copy(src_hbm.at[pl.ds(0, _C)], dstv[d4],
                                  semidx[s]).wait()

        def issue_he(g, e):
            pltpu.async_copy(he_hbm.at[pl.ds(base + g * _C, _C)], hev[e],
                             semhe[e])

        def issue_gather(s, v):
            pltpu.async_copy(hv_hbm.at[srcv[s]], hvv[v], semgv[v])

        def drain_scatter(v):
            pltpu.make_async_copy(he_hbm.at[pl.ds(0, _C)], hvv[v],
                                  semsc[v]).wait()

        def chunk_body(g, q, static):
            v = q % 3
            e = q % 2
            s = q % 2
            d4 = q % 4
            vn = (q + 1) % 3
            sn = (q + 1) % 2

            if static:
                if g >= 2:
                    drain_scatter(vn)
            else:
                @pl.when(g >= 2)
                def _drain():
                    drain_scatter(vn)

            if (not static) or g + 1 < nchunk:
                wait_idx(sn, (q + 1) % 4)
                issue_gather(sn, vn)

            pltpu.make_async_copy(he_hbm.at[pl.ds(0, _C)], hev[e],
                                  semhe[e]).wait()
            pltpu.make_async_copy(he_hbm.at[pl.ds(0, _C)], hvv[v],
                                  semgv[v]).wait()

            @plsc.parallel_loop(0, _C, 1, unroll=2)
            def _mul(r):
                for k in range(_D // 16):
                    sl = pl.ds(k * 16, 16)
                    hvv[v][r, sl] = hvv[v][r, sl] * hev[e][r, sl]

            if static and g == nchunk - 1:
                pltpu.sync_copy(hvv[v], hsh.at[dstv[d4]], add=True)
            else:
                pltpu.async_copy(hvv[v], hsh.at[dstv[d4]], semsc[v], add=True)

            if (not static) or g + 2 < nchunk:
                issue_idx(g + 2, s, (q + 2) % 4)
                issue_he(g + 2, e)

        issue_idx(0, 0, 0)
        issue_idx(1, 1, 1)
        issue_he(0, 0)
        issue_he(1, 1)

        pltpu.sync_copy(init_hbm.at[pl.ds(cid * _N + sid * _RPS, _RPS)],
                        hsh.at[pl.ds(sid * _RPS, _RPS)])

        @pl.when(sid == 0)
        def _init_tail():
            pltpu.sync_copy(init_hbm.at[pl.ds(cid * _N + _NS * _RPS, _TAIL)],
                            hsh.at[pl.ds(_NS * _RPS, _TAIL)])

        plsc.subcore_barrier()
        wait_idx(0, 0)
        issue_gather(0, 0)

        @pl.loop(0, main, step=_STEP)
        def _main(j):
            for q in range(_STEP):
                chunk_body(j + q, q, False)

        for g in range(main, nchunk):
            chunk_body(g, g - main, True)

        drain_scatter((nchunk - 2) % 3)

        plsc.subcore_barrier()
        pltpu.sync_copy(hsh.at[pl.ds(sid * _RPS, _RPS)],
                        out_hbm.at[pl.ds(cid * _N + sid * _RPS, _RPS)])

        @pl.when(sid == 0)
        def _out_tail():
            pltpu.sync_copy(hsh.at[pl.ds(_NS * _RPS, _TAIL)],
                            out_hbm.at[pl.ds(cid * _N + _NS * _RPS, _TAIL)])

    def _body_a(*refs):
        _sc_body(0, *refs)

    def _body_b(*refs):
        _sc_body(_NW * epw, *refs)

    wrap = functools.partial(
        pl.kernel,
        out_type=jax.ShapeDtypeStruct((2 * _N, _D), jnp.float32),
        mesh=plsc.VectorSubcoreMesh(core_axis_name="c", subcore_axis_name="s"),
        scratch_types=[
            pltpu.VMEM((_C,), jnp.int32),
            pltpu.VMEM((_C,), jnp.int32),
            pltpu.VMEM((_C,), jnp.int32),
            pltpu.VMEM((_C,), jnp.int32),
            pltpu.VMEM((_C,), jnp.int32),
            pltpu.VMEM((_C,), jnp.int32),
            pltpu.VMEM((_C, _D), jnp.float32),
            pltpu.VMEM((_C, _D), jnp.float32),
            pltpu.VMEM((_C, _D), jnp.float32),
            pltpu.VMEM((_C, _D), jnp.float32),
            pltpu.VMEM((_C, _D), jnp.float32),
            pltpu.VMEM_SHARED((_N, _D), jnp.float32),
            pltpu.SemaphoreType.DMA,
            pltpu.SemaphoreType.DMA,
            pltpu.SemaphoreType.DMA,
            pltpu.SemaphoreType.DMA,
            pltpu.SemaphoreType.DMA,
            pltpu.SemaphoreType.DMA,
            pltpu.SemaphoreType.DMA,
            pltpu.SemaphoreType.DMA,
            pltpu.SemaphoreType.DMA,
            pltpu.SemaphoreType.DMA,
        ],
    )
    return wrap(_body_a), wrap(_body_b)


_sc_agg_a, _sc_agg_b = _make_sc_aggregate(_EPW // 2)


def kernel(node_feats, edge_feats, edge_index, W_node, gamma_node, beta_node,
           W_edge, gamma_edge, beta_edge, W_out, gamma_out, beta_out):
    hv, src, dst = _node_proj(node_feats, W_node, gamma_node, beta_node,
                              edge_index.astype(jnp.int32), rows_blk=1000)
    half = _E // 2
    he_a = _edge_proj(edge_feats[:half], W_edge, gamma_edge, beta_edge,
                      cols_blk=3200)
    he_b = _edge_proj(edge_feats[half:], W_edge, gamma_edge, beta_edge,
                      cols_blk=3200)
    zero = jnp.zeros((2 * _N, _D), jnp.float32)
    h2a = _sc_agg_a(hv, he_a, src, dst, zero)
    h2b = _sc_agg_b(hv, he_b, src, dst, h2a)
    return _dense_proj(_out_proj_body, [h2b, h2b], W_out, gamma_out, beta_out,
                       rows_blk=1000)


# speedup vs baseline: 83.5617x; 1.0082x over previous
import functools

import jax
import jax.numpy as jnp
from jax import lax
from jax.experimental import pallas as pl
from jax.experimental.pallas import tpu as pltpu
from jax.experimental.pallas import tpu_sc as plsc

_N = 10000
_E = 320000
_D = 128

_NC = 2
_NS = 16
_NW = _NC * _NS
_EPW = _E // _NW
_C = 40
_NCHUNK = _EPW // _C
_STEP = 12
_MAIN = (_NCHUNK - 10) // _STEP * _STEP
_RPS = 624
_TAIL = _N - _NS * _RPS


def _gelu(x):
    return 0.5 * x * (1.0 + lax.erf(x * (2.0 ** -0.5)))


def _ln(y, gamma, beta):
    mu = jnp.mean(y, axis=-1, keepdims=True)
    var = jnp.mean((y - mu) ** 2, axis=-1, keepdims=True)
    return (y - mu) / jnp.sqrt(var + 1e-5) * gamma + beta


def _node_proj_body(x_ref, w_ref, g_ref, b_ref, ei_ref, o_ref, src_ref,
                    dst_ref):
    y = jnp.dot(x_ref[...], w_ref[...], preferred_element_type=jnp.float32)
    y = _gelu(y)
    o_ref[...] = _ln(y, g_ref[...], b_ref[...])
    @pl.when(pl.program_id(0) == 0)
    def _emit_idx():
        ei = ei_ref[...]
        src_ref[...] = ei[0, :]
        dst_ref[...] = ei[1, :]


def _edge_proj_body(xt_ref, w_ref, g_ref, b_ref, o_ref):
    y = lax.dot_general(xt_ref[...], w_ref[...], (((0,), (0,)), ((), ())),
                        preferred_element_type=jnp.float32)
    d = y.shape[1]
    c = jnp.full((d, 1), 1.0 / d, jnp.float32)
    mu = jnp.dot(y, c, preferred_element_type=jnp.float32)
    m2 = jnp.dot(y * y, c, preferred_element_type=jnp.float32)
    var = m2 - mu * mu
    rstd = lax.rsqrt(var + 1e-5)
    o_ref[...] = jnp.exp((y - mu) * rstd * g_ref[...] + b_ref[...])


def _out_proj_body(h0_ref, h1_ref, w_ref, g_ref, b_ref, o_ref):
    h = h0_ref[...] + h1_ref[...]
    y = jnp.dot(h, w_ref[...], preferred_element_type=jnp.float32)
    y = _gelu(y)
    o_ref[...] = _ln(y, g_ref[...], b_ref[...])


def _edge_proj(x, w, g, b, cols_blk):
    xt = x.T
    k, n = xt.shape
    d = w.shape[1]
    nblk = n // cols_blk
    return pl.pallas_call(
        _edge_proj_body,
        grid=(nblk,),
        in_specs=[
            pl.BlockSpec((k, cols_blk), lambda i: (0, i)),
            pl.BlockSpec((k, d), lambda i: (0, 0)),
            pl.BlockSpec((1, d), lambda i: (0, 0)),
            pl.BlockSpec((1, d), lambda i: (0, 0)),
        ],
        out_specs=pl.BlockSpec((cols_blk, d), lambda i: (i, 0)),
        out_shape=jax.ShapeDtypeStruct((n, d), jnp.float32),
    )(xt, w, g.reshape(1, d), b.reshape(1, d))


def _node_proj(x, w, g, b, edge_index, rows_blk):
    n, k = x.shape
    d = w.shape[1]
    nblk = n // rows_blk
    return pl.pallas_call(
        _node_proj_body,
        grid=(nblk,),
        in_specs=[
            pl.BlockSpec((rows_blk, k), lambda i: (i, 0)),
            pl.BlockSpec((k, d), lambda i: (0, 0)),
            pl.BlockSpec((1, d), lambda i: (0, 0)),
            pl.BlockSpec((1, d), lambda i: (0, 0)),
            pl.BlockSpec((2, _E), lambda i: (0, 0)),
        ],
        out_specs=[
            pl.BlockSpec((rows_blk, d), lambda i: (i, 0)),
            pl.BlockSpec((_E,), lambda i: (0,)),
            pl.BlockSpec((_E,), lambda i: (0,)),
        ],
        out_shape=[
            jax.ShapeDtypeStruct((n, d), jnp.float32),
            jax.ShapeDtypeStruct((_E,), jnp.int32),
            jax.ShapeDtypeStruct((_E,), jnp.int32),
        ],
    )(x, w, g.reshape(1, d), b.reshape(1, d), edge_index)


def _dense_proj(body, x_parts, w, g, b, rows_blk):
    n = x_parts[0].shape[0] // 2
    k = x_parts[0].shape[1]
    d = w.shape[1]
    nblk = n // rows_blk
    in_specs = [
        pl.BlockSpec((rows_blk, k), lambda i: (i, 0)),
        pl.BlockSpec((rows_blk, k), lambda i, _n=nblk: (i + _n, 0)),
    ]
    args = list(x_parts)
    return pl.pallas_call(
        body,
        grid=(nblk,),
        in_specs=in_specs + [
            pl.BlockSpec((k, d), lambda i: (0, 0)),
            pl.BlockSpec((1, d), lambda i: (0, 0)),
            pl.BlockSpec((1, d), lambda i: (0, 0)),
        ],
        out_specs=pl.BlockSpec((rows_blk, d), lambda i: (i, 0)),
        out_shape=jax.ShapeDtypeStruct((n, d), jnp.float32),
    )(*args, w, g.reshape(1, d), b.reshape(1, d))


def _make_sc_aggregate(epw):
    nchunk = epw // _C
    main = (nchunk - 2) // _STEP * _STEP

    def _sc_body(ebase0, hv_hbm, he_hbm, src_hbm, dst_hbm, init_hbm, out_hbm,
                 srcv0, srcv1, dstv0, dstv1, dstv2, dstv3,
                 hev0, hev1, hvv0, hvv1, hvv2, hsh,
                 semidx0, semidx1, semhe0, semhe1,
                 semgv0, semgv1, semgv2, semsc0, semsc1, semsc2):
        srcv = (srcv0, srcv1)
        dstv = (dstv0, dstv1, dstv2, dstv3)
        hev = (hev0, hev1)
        hvv = (hvv0, hvv1, hvv2)
        semidx = (semidx0, semidx1)
        semhe = (semhe0, semhe1)
        semgv = (semgv0, semgv1, semgv2)
        semsc = (semsc0, semsc1, semsc2)

        cid = lax.axis_index("c")
        sid = lax.axis_index("s")
        wid = sid * _NC + cid
        base = wid * epw
        ibase = ebase0 + base

        def issue_idx(g, s, d4):
            off = pl.ds(ibase + g * _C, _C)
            pltpu.async_copy(src_hbm.at[off], srcv[s], semidx[s])
            pltpu.async_copy(dst_hbm.at[off], dstv[d4], semidx[s])

        def wait_idx(s, d4):
            pltpu.make_async_copy(src_hbm.at[pl.ds(0, _C)], srcv[s],
                                  semidx[s]).wait()
            pltpu.make_async_copy(src_hbm.at[pl.ds(0, _C)], dstv[d4],
                                  semidx[s]).wait()

        def issue_he(g, e):
            pltpu.async_copy(he_hbm.at[pl.ds(base + g * _C, _C)], hev[e],
                             semhe[e])

        def issue_gather(s, v):
            pltpu.async_copy(hv_hbm.at[srcv[s]], hvv[v], semgv[v])

        def drain_scatter(v):
            pltpu.make_async_copy(he_hbm.at[pl.ds(0, _C)], hvv[v],
                                  semsc[v]).wait()

        def chunk_body(g, q, static):
            v = q % 3
            e = q % 2
            s = q % 2
            d4 = q % 4
            vn = (q + 1) % 3
            sn = (q + 1) % 2

            if static:
                if g >= 2:
                    drain_scatter(vn)
            else:
                @pl.when(g >= 2)
                def _drain():
                    drain_scatter(vn)

            if (not static) or g + 1 < nchunk:
                wait_idx(sn, (q + 1) % 4)
                issue_gather(sn, vn)

            pltpu.make_async_copy(he_hbm.at[pl.ds(0, _C)], hev[e],
                                  semhe[e]).wait()
            pltpu.make_async_copy(he_hbm.at[pl.ds(0, _C)], hvv[v],
                                  semgv[v]).wait()

            @plsc.parallel_loop(0, _C, 1, unroll=4)
            def _mul(r):
                for k in range(_D // 16):
                    sl = pl.ds(k * 16, 16)
                    hvv[v][r, sl] = hvv[v][r, sl] * hev[e][r, sl]

            if static and g == nchunk - 1:
                pltpu.sync_copy(hvv[v], hsh.at[dstv[d4]], add=True)
            else:
                pltpu.async_copy(hvv[v], hsh.at[dstv[d4]], semsc[v], add=True)

            if (not static) or g + 2 < nchunk:
                issue_idx(g + 2, s, (q + 2) % 4)
                issue_he(g + 2, e)

        issue_idx(0, 0, 0)
        issue_idx(1, 1, 1)
        issue_he(0, 0)
        issue_he(1, 1)

        pltpu.sync_copy(init_hbm.at[pl.ds(cid * _N + sid * _RPS, _RPS)],
                        hsh.at[pl.ds(sid * _RPS, _RPS)])

        @pl.when(sid == 0)
        def _init_tail():
            pltpu.sync_copy(init_hbm.at[pl.ds(cid * _N + _NS * _RPS, _TAIL)],
                            hsh.at[pl.ds(_NS * _RPS, _TAIL)])

        plsc.subcore_barrier()
        wait_idx(0, 0)
        issue_gather(0, 0)

        @pl.loop(0, main, step=_STEP)
        def _main(j):
            for q in range(_STEP):
                chunk_body(j + q, q, False)

        for g in range(main, nchunk):
            chunk_body(g, g - main, True)

        drain_scatter((nchunk - 2) % 3)

        plsc.subcore_barrier()
        pltpu.sync_copy(hsh.at[pl.ds(sid * _RPS, _RPS)],
                        out_hbm.at[pl.ds(cid * _N + sid * _RPS, _RPS)])

        @pl.when(sid == 0)
        def _out_tail():
            pltpu.sync_copy(hsh.at[pl.ds(_NS * _RPS, _TAIL)],
                            out_hbm.at[pl.ds(cid * _N + _NS * _RPS, _TAIL)])

    def _body_a(*refs):
        _sc_body(0, *refs)

    def _body_b(*refs):
        _sc_body(_NW * epw, *refs)

    wrap = functools.partial(
        pl.kernel,
        out_type=jax.ShapeDtypeStruct((2 * _N, _D), jnp.float32),
        mesh=plsc.VectorSubcoreMesh(core_axis_name="c", subcore_axis_name="s"),
        scratch_types=[
            pltpu.VMEM((_C,), jnp.int32),
            pltpu.VMEM((_C,), jnp.int32),
            pltpu.VMEM((_C,), jnp.int32),
            pltpu.VMEM((_C,), jnp.int32),
            pltpu.VMEM((_C,), jnp.int32),
            pltpu.VMEM((_C,), jnp.int32),
            pltpu.VMEM((_C, _D), jnp.float32),
            pltpu.VMEM((_C, _D), jnp.float32),
            pltpu.VMEM((_C, _D), jnp.float32),
            pltpu.VMEM((_C, _D), jnp.float32),
            pltpu.VMEM((_C, _D), jnp.float32),
            pltpu.VMEM_SHARED((_N, _D), jnp.float32),
            pltpu.SemaphoreType.DMA,
            pltpu.SemaphoreType.DMA,
            pltpu.SemaphoreType.DMA,
            pltpu.SemaphoreType.DMA,
            pltpu.SemaphoreType.DMA,
            pltpu.SemaphoreType.DMA,
            pltpu.SemaphoreType.DMA,
            pltpu.SemaphoreType.DMA,
            pltpu.SemaphoreType.DMA,
            pltpu.SemaphoreType.DMA,
        ],
    )
    return wrap(_body_a), wrap(_body_b)


_sc_agg_a, _sc_agg_b = _make_sc_aggregate(_EPW // 2)


def kernel(node_feats, edge_feats, edge_index, W_node, gamma_node, beta_node,
           W_edge, gamma_edge, beta_edge, W_out, gamma_out, beta_out):
    hv, src, dst = _node_proj(node_feats, W_node, gamma_node, beta_node,
                              edge_index.astype(jnp.int32), rows_blk=1000)
    half = _E // 2
    he_a = _edge_proj(edge_feats[:half], W_edge, gamma_edge, beta_edge,
                      cols_blk=3200)
    he_b = _edge_proj(edge_feats[half:], W_edge, gamma_edge, beta_edge,
                      cols_blk=3200)
    zero = jnp.zeros((2 * _N, _D), jnp.float32)
    h2a = _sc_agg_a(hv, he_a, src, dst, zero)
    h2b = _sc_agg_b(hv, he_b, src, dst, h2a)
    return _dense_proj(_out_proj_body, [h2b, h2b], W_out, gamma_out, beta_out,
                       rows_blk=1000)


# speedup vs baseline: 86.1405x; 1.0309x over previous
import functools

import jax
import jax.numpy as jnp
from jax import lax
from jax.experimental import pallas as pl
from jax.experimental.pallas import tpu as pltpu
from jax.experimental.pallas import tpu_sc as plsc

_N = 10000
_E = 320000
_D = 128

_NC = 2
_NS = 16
_NW = _NC * _NS
_EPW = _E // _NW
_C = 40
_NCHUNK = _EPW // _C
_STEP = 12
_RPS = 624
_TAIL = _N - _NS * _RPS


def _gelu(x):
    return 0.5 * x * (1.0 + lax.erf(x * (2.0 ** -0.5)))


def _ln(y, gamma, beta):
    mu = jnp.mean(y, axis=-1, keepdims=True)
    var = jnp.mean((y - mu) ** 2, axis=-1, keepdims=True)
    return (y - mu) / jnp.sqrt(var + 1e-5) * gamma + beta


def _node_proj_body(x_ref, w_ref, g_ref, b_ref, ei_ref, o_ref, src_ref,
                    dst_ref):
    y = jnp.dot(x_ref[...], w_ref[...], preferred_element_type=jnp.float32)
    y = _gelu(y)
    o_ref[...] = _ln(y, g_ref[...], b_ref[...])
    @pl.when(pl.program_id(0) == 0)
    def _emit_idx():
        ei = ei_ref[...]
        src_ref[...] = ei[0, :]
        dst_ref[...] = ei[1, :]


def _edge_proj_body(xt_ref, w_ref, g_ref, b_ref, o_ref):
    y = lax.dot_general(xt_ref[...], w_ref[...], (((0,), (0,)), ((), ())),
                        preferred_element_type=jnp.float32)
    d = y.shape[1]
    c = jnp.full((d, 1), 1.0 / d, jnp.float32)
    mu = jnp.dot(y, c, preferred_element_type=jnp.float32)
    m2 = jnp.dot(y * y, c, preferred_element_type=jnp.float32)
    var = m2 - mu * mu
    rstd = lax.rsqrt(var + 1e-5)
    o_ref[...] = jnp.exp((y - mu) * rstd * g_ref[...] + b_ref[...])


def _out_proj_body(h0_ref, h1_ref, w_ref, g_ref, b_ref, o_ref):
    h = h0_ref[...] + h1_ref[...]
    y = jnp.dot(h, w_ref[...], preferred_element_type=jnp.float32)
    y = _gelu(y)
    o_ref[...] = _ln(y, g_ref[...], b_ref[...])


def _edge_proj(x, w, g, b, cols_blk):
    xt = x.T
    k, n = xt.shape
    d = w.shape[1]
    nblk = n // cols_blk
    return pl.pallas_call(
        _edge_proj_body,
        grid=(nblk,),
        in_specs=[
            pl.BlockSpec((k, cols_blk), lambda i: (0, i)),
            pl.BlockSpec((k, d), lambda i: (0, 0)),
            pl.BlockSpec((1, d), lambda i: (0, 0)),
            pl.BlockSpec((1, d), lambda i: (0, 0)),
        ],
        out_specs=pl.BlockSpec((cols_blk, d), lambda i: (i, 0)),
        out_shape=jax.ShapeDtypeStruct((n, d), jnp.float32),
    )(xt, w, g.reshape(1, d), b.reshape(1, d))


def _node_proj(x, w, g, b, edge_index, rows_blk):
    n, k = x.shape
    d = w.shape[1]
    nblk = n // rows_blk
    return pl.pallas_call(
        _node_proj_body,
        grid=(nblk,),
        in_specs=[
            pl.BlockSpec((rows_blk, k), lambda i: (i, 0)),
            pl.BlockSpec((k, d), lambda i: (0, 0)),
            pl.BlockSpec((1, d), lambda i: (0, 0)),
            pl.BlockSpec((1, d), lambda i: (0, 0)),
            pl.BlockSpec((2, _E), lambda i: (0, 0)),
        ],
        out_specs=[
            pl.BlockSpec((rows_blk, d), lambda i: (i, 0)),
            pl.BlockSpec((_E,), lambda i: (0,)),
            pl.BlockSpec((_E,), lambda i: (0,)),
        ],
        out_shape=[
            jax.ShapeDtypeStruct((n, d), jnp.float32),
            jax.ShapeDtypeStruct((_E,), jnp.int32),
            jax.ShapeDtypeStruct((_E,), jnp.int32),
        ],
    )(x, w, g.reshape(1, d), b.reshape(1, d), edge_index)


def _dense_proj(body, x_parts, w, g, b, rows_blk):
    n = x_parts[0].shape[0] // 2
    k = x_parts[0].shape[1]
    d = w.shape[1]
    nblk = n // rows_blk
    in_specs = [
        pl.BlockSpec((rows_blk, k), lambda i: (i, 0)),
        pl.BlockSpec((rows_blk, k), lambda i, _n=nblk: (i + _n, 0)),
    ]
    args = list(x_parts)
    return pl.pallas_call(
        body,
        grid=(nblk,),
        in_specs=in_specs + [
            pl.BlockSpec((k, d), lambda i: (0, 0)),
            pl.BlockSpec((1, d), lambda i: (0, 0)),
            pl.BlockSpec((1, d), lambda i: (0, 0)),
        ],
        out_specs=pl.BlockSpec((rows_blk, d), lambda i: (i, 0)),
        out_shape=jax.ShapeDtypeStruct((n, d), jnp.float32),
    )(*args, w, g.reshape(1, d), b.reshape(1, d))


def _make_sc_aggregate(epw):
    nchunk = epw // _C
    main = (nchunk - 2) // _STEP * _STEP

    def _sc_body(ebase0, hv_hbm, he_hbm, src_hbm, dst_hbm, init_hbm, out_hbm,
                 srcv0, srcv1, dstv0, dstv1, dstv2, dstv3,
                 hev0, hev1, hvv0, hvv1, hvv2, hsh,
                 semidx0, semidx1, semhe0, semhe1,
                 semgv0, semgv1, semgv2, semsc0, semsc1, semsc2):
        srcv = (srcv0, srcv1)
        dstv = (dstv0, dstv1, dstv2, dstv3)
        hev = (hev0, hev1)
        hvv = (hvv0, hvv1, hvv2)
        semidx = (semidx0, semidx1)
        semhe = (semhe0, semhe1)
        semgv = (semgv0, semgv1, semgv2)
        semsc = (semsc0, semsc1, semsc2)

        cid = lax.axis_index("c")
        sid = lax.axis_index("s")
        wid = sid * _NC + cid
        base = wid * epw
        ibase = ebase0 + base

        def issue_idx(g, s, d4):
            off = pl.ds(ibase + g * _C, _C)
            pltpu.async_copy(src_hbm.at[off], srcv[s], semidx[s])
            pltpu.async_copy(dst_hbm.at[off], dstv[d4], semidx[s])

        def wait_idx(s, d4):
            pltpu.make_async_copy(src_hbm.at[pl.ds(0, _C)], srcv[s],
                                  semidx[s]).wait()
            pltpu.make_async_copy(src_hbm.at[pl.ds(0, _C)], dstv[d4],
                                  semidx[s]).wait()

        def issue_he(g, e):
            pltpu.async_copy(he_hbm.at[pl.ds(base + g * _C, _C)], hev[e],
                             semhe[e])

        def issue_gather(s, v):
            pltpu.async_copy(hv_hbm.at[srcv[s]], hvv[v], semgv[v])

        def drain_scatter(v):
            pltpu.make_async_copy(he_hbm.at[pl.ds(0, _C)], hvv[v],
                                  semsc[v]).wait()

        def chunk_body(g, q, static):
            v = q % 3
            e = q % 2
            s = q % 2
            d4 = q % 4
            vn = (q + 1) % 3
            sn = (q + 1) % 2

            if static:
                if g >= 2:
                    drain_scatter(vn)
            else:
                @pl.when(g >= 2)
                def _drain():
                    drain_scatter(vn)

            if (not static) or g + 1 < nchunk:
                wait_idx(sn, (q + 1) % 4)
                issue_gather(sn, vn)

            pltpu.make_async_copy(he_hbm.at[pl.ds(0, _C)], hev[e],
                                  semhe[e]).wait()
            pltpu.make_async_copy(he_hbm.at[pl.ds(0, _C)], hvv[v],
                                  semgv[v]).wait()

            @plsc.parallel_loop(0, _C, 1, unroll=4)
            def _mul(r):
                for k in range(_D // 16):
                    sl = pl.ds(k * 16, 16)
                    hvv[v][r, sl] = hvv[v][r, sl] * hev[e][r, sl]

            if static and g == nchunk - 1:
                pltpu.sync_copy(hvv[v], hsh.at[dstv[d4]], add=True)
            else:
                pltpu.async_copy(hvv[v], hsh.at[dstv[d4]], semsc[v], add=True)

            if (not static) or g + 2 < nchunk:
                issue_idx(g + 2, s, (q + 2) % 4)
                issue_he(g + 2, e)

        issue_idx(0, 0, 0)
        issue_idx(1, 1, 1)
        issue_he(0, 0)
        issue_he(1, 1)

        pltpu.sync_copy(init_hbm.at[pl.ds(cid * _N + sid * _RPS, _RPS)],
                        hsh.at[pl.ds(sid * _RPS, _RPS)])

        @pl.when(sid == 0)
        def _init_tail():
            pltpu.sync_copy(init_hbm.at[pl.ds(cid * _N + _NS * _RPS, _TAIL)],
                            hsh.at[pl.ds(_NS * _RPS, _TAIL)])

        plsc.subcore_barrier()
        wait_idx(0, 0)
        issue_gather(0, 0)

        @pl.loop(0, main, step=_STEP)
        def _main(j):
            for q in range(_STEP):
                chunk_body(j + q, q, False)

        for g in range(main, nchunk):
            chunk_body(g, g - main, True)

        drain_scatter((nchunk - 2) % 3)

        plsc.subcore_barrier()
        pltpu.sync_copy(hsh.at[pl.ds(sid * _RPS, _RPS)],
                        out_hbm.at[pl.ds(cid * _N + sid * _RPS, _RPS)])

        @pl.when(sid == 0)
        def _out_tail():
            pltpu.sync_copy(hsh.at[pl.ds(_NS * _RPS, _TAIL)],
                            out_hbm.at[pl.ds(cid * _N + _NS * _RPS, _TAIL)])

    def _body_a(*refs):
        _sc_body(0, *refs)

    def _body_b(*refs):
        _sc_body(_NW * epw, *refs)

    wrap = functools.partial(
        pl.kernel,
        out_type=jax.ShapeDtypeStruct((2 * _N, _D), jnp.float32),
        mesh=plsc.VectorSubcoreMesh(core_axis_name="c", subcore_axis_name="s"),
        scratch_types=[
            pltpu.VMEM((_C,), jnp.int32),
            pltpu.VMEM((_C,), jnp.int32),
            pltpu.VMEM((_C,), jnp.int32),
            pltpu.VMEM((_C,), jnp.int32),
            pltpu.VMEM((_C,), jnp.int32),
            pltpu.VMEM((_C,), jnp.int32),
            pltpu.VMEM((_C, _D), jnp.float32),
            pltpu.VMEM((_C, _D), jnp.float32),
            pltpu.VMEM((_C, _D), jnp.float32),
            pltpu.VMEM((_C, _D), jnp.float32),
            pltpu.VMEM((_C, _D), jnp.float32),
            pltpu.VMEM_SHARED((_N, _D), jnp.float32),
            pltpu.SemaphoreType.DMA,
            pltpu.SemaphoreType.DMA,
            pltpu.SemaphoreType.DMA,
            pltpu.SemaphoreType.DMA,
            pltpu.SemaphoreType.DMA,
            pltpu.SemaphoreType.DMA,
            pltpu.SemaphoreType.DMA,
            pltpu.SemaphoreType.DMA,
            pltpu.SemaphoreType.DMA,
            pltpu.SemaphoreType.DMA,
        ],
    )
    return wrap(_body_a), wrap(_body_b)


_sc_agg_a, _sc_agg_b = _make_sc_aggregate(_EPW // 2)


def kernel(node_feats, edge_feats, edge_index, W_node, gamma_node, beta_node,
           W_edge, gamma_edge, beta_edge, W_out, gamma_out, beta_out):
    hv, src, dst = _node_proj(node_feats, W_node, gamma_node, beta_node,
                              edge_index.astype(jnp.int32), rows_blk=2000)
    half = _E // 2
    he_a = _edge_proj(edge_feats[:half], W_edge, gamma_edge, beta_edge,
                      cols_blk=6400)
    he_b = _edge_proj(edge_feats[half:], W_edge, gamma_edge, beta_edge,
                      cols_blk=6400)
    zero = jnp.zeros((2 * _N, _D), jnp.float32)
    h2a = _sc_agg_a(hv, he_a, src, dst, zero)
    h2b = _sc_agg_b(hv, he_b, src, dst, h2a)
    return _dense_proj(_out_proj_body, [h2b, h2b], W_out, gamma_out, beta_out,
                       rows_blk=2000)
